# baseline (speedup 1.0000x reference)
import functools

import jax
import jax.numpy as jnp
from jax.experimental import pallas as pl


_BM = 256


def _linear_kernel(x_ref, w_ref, b_ref, o_ref):
    acc = jax.lax.dot_general(
        x_ref[...].astype(jnp.bfloat16),
        w_ref[...],
        dimension_numbers=(((1,), (1,)), ((), ())),
        preferred_element_type=jnp.float32,
    )
    o_ref[...] = acc + b_ref[...]


@jax.jit
def kernel(x, weight, bias):
    m, k = x.shape
    n = weight.shape[0]
    bias2d = bias.reshape(1, n)
    grid = (m // _BM,)
    return pl.pallas_call(
        _linear_kernel,
        grid=grid,
        in_specs=[
            pl.BlockSpec((_BM, k), lambda i: (i, 0)),
            pl.BlockSpec((n, k), lambda i: (0, 0)),
            pl.BlockSpec((1, n), lambda i: (0, 0)),
        ],
        out_specs=pl.BlockSpec((_BM, n), lambda i: (i, 0)),
        out_shape=jax.ShapeDtypeStruct((m, n), jnp.float32),
    )(x, weight, bias2d)


# speedup vs baseline: 1.2848x; 1.2848x over previous
import jax
import jax.numpy as jnp
from jax.experimental import pallas as pl
from jax.experimental.pallas import tpu as pltpu


_BM = 512


def _linear_kernel(x_ref, w_ref, b_ref, o_ref, w_bf):
    @pl.when(pl.program_id(0) == 0)
    def _cast_weight():
        w_bf[...] = w_ref[...].astype(jnp.bfloat16)

    acc = jax.lax.dot_general(
        x_ref[...].astype(jnp.bfloat16),
        w_bf[...],
        dimension_numbers=(((1,), (1,)), ((), ())),
        preferred_element_type=jnp.float32,
    )
    o_ref[...] = acc + b_ref[...]


@jax.jit
def kernel(x, weight, bias):
    m, k = x.shape
    n = weight.shape[0]
    bias2d = bias.reshape(1, n)
    grid = (m // _BM,)
    return pl.pallas_call(
        _linear_kernel,
        grid=grid,
        in_specs=[
            pl.BlockSpec((_BM, k), lambda i: (i, 0)),
            pl.BlockSpec((n, k), lambda i: (0, 0)),
            pl.BlockSpec((1, n), lambda i: (0, 0)),
        ],
        out_specs=pl.BlockSpec((_BM, n), lambda i: (i, 0)),
        out_shape=jax.ShapeDtypeStruct((m, n), jnp.float32),
        scratch_shapes=[pltpu.VMEM((n, k), jnp.bfloat16)],
    )(x, weight, bias2d)


# speedup vs baseline: 1.5311x; 1.1917x over previous
import jax
import jax.numpy as jnp
from jax.experimental import pallas as pl
from jax.experimental.pallas import tpu as pltpu


_BM = 2048


def _linear_kernel(x_ref, w_ref, b_ref, o_ref, w_bf):
    @pl.when(pl.program_id(0) == 0)
    def _cast_weight():
        w_bf[...] = w_ref[...].astype(jnp.bfloat16)

    acc = jax.lax.dot_general(
        x_ref[...].astype(jnp.bfloat16),
        w_bf[...],
        dimension_numbers=(((1,), (1,)), ((), ())),
        preferred_element_type=jnp.float32,
    )
    o_ref[...] = acc + b_ref[...]


@jax.jit
def kernel(x, weight, bias):
    m, k = x.shape
    n = weight.shape[0]
    bias2d = bias.reshape(1, n)
    grid = (m // _BM,)
    return pl.pallas_call(
        _linear_kernel,
        grid=grid,
        in_specs=[
            pl.BlockSpec((_BM, k), lambda i: (i, 0)),
            pl.BlockSpec((n, k), lambda i: (0, 0)),
            pl.BlockSpec((1, n), lambda i: (0, 0)),
        ],
        out_specs=pl.BlockSpec((_BM, n), lambda i: (i, 0)),
        out_shape=jax.ShapeDtypeStruct((m, n), jnp.float32),
        scratch_shapes=[pltpu.VMEM((n, k), jnp.bfloat16)],
    )(x, weight, bias2d)
